# baseline (speedup 1.0000x reference)
import jax
import jax.numpy as jnp
from jax.experimental import pallas as pl
from jax.experimental.pallas import tpu as pltpu

_T, _D, _E, _F = 2048, 768, 8, 3072
_K = 512


def _tdot(a, b):
    return jax.lax.dot_general(a, b, (((0,), (0,)), ((), ())),
                               preferred_element_type=jnp.float32)


def _router_t(x, gate_w):
    scores = jax.lax.dot_general(gate_w, x, (((0,), (1,)), ((), ())),
                                 preferred_element_type=jnp.float32)
    m = jnp.max(scores, axis=0, keepdims=True)
    ex = jnp.exp(scores - m)
    probs = ex / jnp.sum(ex, axis=0, keepdims=True)
    pbits = jax.lax.bitcast_convert_type(probs, jnp.int32)

    def vstep(_, carry):
        lo, hi = carry
        mid = (lo + hi) // 2
        cnt = jnp.sum((pbits >= mid).astype(jnp.int32), axis=1, keepdims=True)
        big = cnt >= _K
        return jnp.where(big, mid, lo), jnp.where(big, hi, mid)

    lo0 = jnp.zeros((_E, 1), jnp.int32)
    hi0 = jnp.full((_E, 1), 0x7F800000, jnp.int32)
    v, _ = jax.lax.fori_loop(0, 31, vstep, (lo0, hi0))

    gt = pbits > v
    eq = pbits == v
    idx = jax.lax.broadcasted_iota(jnp.int32, (_E, _T), 1)

    def tstep(_, carry):
        lo, hi = carry
        mid = (lo + hi) // 2
        cnt = jnp.sum((gt | (eq & (idx < mid))).astype(jnp.int32),
                      axis=1, keepdims=True)
        big = cnt >= _K
        return jnp.where(big, lo, mid), jnp.where(big, mid, hi)

    tlo0 = jnp.zeros((_E, 1), jnp.int32)
    thi0 = jnp.full((_E, 1), _T, jnp.int32)
    _, tthr = jax.lax.fori_loop(0, 11, tstep, (tlo0, thi0))

    sel = gt | (eq & (idx < tthr))
    gsel = jnp.where(sel, probs, 0.0)

    rank = sel.astype(jnp.float32)
    s = 1
    while s < _T:
        rank = rank + jnp.concatenate(
            [jnp.zeros((_E, s), jnp.float32), rank[:, :-s]], axis=1)
        s *= 2
    rank = rank - sel.astype(jnp.float32)
    return gsel, rank


def _moe_kernel(x_ref, gw_ref, w1_ref, w2_ref, out_ref, gr_ref, xc_ref,
                pg_ref, acc_ref):
    s = pl.program_id(0)

    @pl.when(s == 0)
    def _():
        gsel, rank = _router_t(x_ref[0], gw_ref[...])
        gr_ref[...] = jnp.concatenate([gsel, rank], axis=0)
        acc_ref[...] = jnp.zeros((_T, _D), jnp.bfloat16)
        xc_ref[1] = jnp.zeros((_K, _D), jnp.bfloat16)
        pg_ref[1] = jnp.zeros((_K, _T), jnp.bfloat16)

    sd = jnp.minimum(s, _E - 1)
    g_row = gr_ref[pl.ds(sd, 1), :]
    r_row = gr_ref[pl.ds(sd + _E, 1), :]
    sel_row = g_row > 0.0
    kio = jax.lax.broadcasted_iota(jnp.int32, (_K, _T), 0)
    r_int = r_row.astype(jnp.int32)
    p32 = jnp.where((kio == r_int) & sel_row, 1.0, 0.0)
    p = p32.astype(jnp.bfloat16)
    pg_ref[sd % 2] = (p32 * g_row).astype(jnp.bfloat16)
    xc_new = jnp.dot(p, x_ref[0].astype(jnp.bfloat16),
                     preferred_element_type=jnp.float32)
    xc_ref[sd % 2] = xc_new.astype(jnp.bfloat16)

    xc = xc_ref[(s + 1) % 2]
    nc = 4
    fc = _F // nc
    h_parts = []
    for j in range(nc):
        w1j = w1_ref[0, :, j * fc:(j + 1) * fc].astype(jnp.bfloat16)
        hj = jnp.dot(xc, w1j, preferred_element_type=jnp.float32)
        h_parts.append(jax.nn.gelu(hj).astype(jnp.bfloat16))
    h = jnp.concatenate(h_parts, axis=1)
    w2 = w2_ref[0].astype(jnp.bfloat16)
    h2 = jnp.dot(h, w2, preferred_element_type=jnp.float32)
    pg = pg_ref[(s + 1) % 2]
    acc_ref[...] += _tdot(pg, h2.astype(jnp.bfloat16)).astype(jnp.bfloat16)

    @pl.when(s == _E)
    def _():
        out_ref[0] = acc_ref[...]


def kernel(x, w1, w2, gate_w):
    x = x.astype(jnp.float32)
    out = pl.pallas_call(
        _moe_kernel,
        grid=(_E + 1,),
        in_specs=[
            pl.BlockSpec((1, _T, _D), lambda s: (0, 0, 0)),
            pl.BlockSpec((_D, _E), lambda s: (0, 0)),
            pl.BlockSpec((1, _D, _F), lambda s: (jnp.maximum(s - 1, 0), 0, 0)),
            pl.BlockSpec((1, _F, _D), lambda s: (jnp.maximum(s - 1, 0), 0, 0)),
        ],
        out_specs=pl.BlockSpec((1, _T, _D), lambda s: (0, 0, 0)),
        out_shape=jax.ShapeDtypeStruct((1, _T, _D), jnp.bfloat16),
        scratch_shapes=[
            pltpu.VMEM((2 * _E, _T), jnp.float32),
            pltpu.VMEM((2, _K, _D), jnp.bfloat16),
            pltpu.VMEM((2, _K, _T), jnp.bfloat16),
            pltpu.VMEM((_T, _D), jnp.bfloat16),
        ],
        compiler_params=pltpu.CompilerParams(
            dimension_semantics=("arbitrary",),
            vmem_limit_bytes=64 * 1024 * 1024,
        ),
    )(x, gate_w, w1, w2)
    return out


# speedup vs baseline: 1.1305x; 1.1305x over previous
import jax
import jax.numpy as jnp
from jax.experimental import pallas as pl
from jax.experimental.pallas import tpu as pltpu

_T, _D, _E, _F = 2048, 768, 8, 3072
_K = 512


def _tdot(a, b):
    return jax.lax.dot_general(a, b, (((0,), (0,)), ((), ())),
                               preferred_element_type=jnp.float32)


def _router_t(x, gate_w):
    scores = jax.lax.dot_general(gate_w, x, (((0,), (1,)), ((), ())),
                                 preferred_element_type=jnp.float32)
    m = jnp.max(scores, axis=0, keepdims=True)
    ex = jnp.exp(scores - m)
    probs = ex / jnp.sum(ex, axis=0, keepdims=True)
    pbits = jax.lax.bitcast_convert_type(probs, jnp.int32)

    def vstep(_, carry):
        lo, hi = carry
        mid = (lo + hi) // 2
        cnt = jnp.sum((pbits >= mid).astype(jnp.int32), axis=1, keepdims=True)
        big = cnt >= _K
        return jnp.where(big, mid, lo), jnp.where(big, hi, mid)

    lo0 = jnp.zeros((_E, 1), jnp.int32)
    hi0 = jnp.full((_E, 1), 0x7F800000, jnp.int32)
    v, _ = jax.lax.fori_loop(0, 31, vstep, (lo0, hi0))

    gt = pbits > v
    eq = pbits == v
    idx = jax.lax.broadcasted_iota(jnp.int32, (_E, _T), 1)

    def tstep(_, carry):
        lo, hi = carry
        mid = (lo + hi) // 2
        cnt = jnp.sum((gt | (eq & (idx < mid))).astype(jnp.int32),
                      axis=1, keepdims=True)
        big = cnt >= _K
        return jnp.where(big, lo, mid), jnp.where(big, mid, hi)

    tlo0 = jnp.zeros((_E, 1), jnp.int32)
    thi0 = jnp.full((_E, 1), _T, jnp.int32)
    _, tthr = jax.lax.fori_loop(0, 11, tstep, (tlo0, thi0))

    sel = gt | (eq & (idx < tthr))
    gsel = jnp.where(sel, probs, 0.0)

    rank = sel.astype(jnp.float32)
    s = 1
    while s < _T:
        rank = rank + jnp.concatenate(
            [jnp.zeros((_E, s), jnp.float32), rank[:, :-s]], axis=1)
        s *= 2
    rank = rank - sel.astype(jnp.float32)
    return gsel, rank


def _moe_kernel(x_ref, gw_ref, w1_ref, w2_ref, out_ref, gr_ref, xc_ref,
                pg_ref, acc_ref):
    s = pl.program_id(0)

    @pl.when(s == 0)
    def _():
        gsel, rank = _router_t(x_ref[0], gw_ref[...])
        gr_ref[...] = jnp.concatenate([gsel, rank], axis=0)
        acc_ref[...] = jnp.zeros((_T, _D), jnp.bfloat16)

    @pl.when(s < _E)
    def _():
        g_row = gr_ref[pl.ds(s, 1), :]
        r_row = gr_ref[pl.ds(s + _E, 1), :]
        sel_row = g_row > 0.0
        kio = jax.lax.broadcasted_iota(jnp.int32, (_K, _T), 0)
        r_int = r_row.astype(jnp.int32)
        p32 = jnp.where((kio == r_int) & sel_row, 1.0, 0.0)
        p = p32.astype(jnp.bfloat16)
        pg_ref[s % 2] = (p32 * g_row).astype(jnp.bfloat16)
        xc = jnp.dot(p, x_ref[0].astype(jnp.bfloat16),
                     preferred_element_type=jnp.float32)
        xc_ref[s % 2] = xc.astype(jnp.bfloat16)

    @pl.when(s > 0)
    def _():
        xc = xc_ref[(s + 1) % 2]
        nc = 4
        fc = _F // nc
        h_parts = []
        for j in range(nc):
            w1j = w1_ref[0, :, j * fc:(j + 1) * fc].astype(jnp.bfloat16)
            hj = jnp.dot(xc, w1j, preferred_element_type=jnp.float32)
            h_parts.append(jax.nn.gelu(hj).astype(jnp.bfloat16))
        h = jnp.concatenate(h_parts, axis=1)
        w2 = w2_ref[0].astype(jnp.bfloat16)
        h2 = jnp.dot(h, w2, preferred_element_type=jnp.float32)
        pg = pg_ref[(s + 1) % 2]
        acc_ref[...] += _tdot(pg, h2.astype(jnp.bfloat16)).astype(jnp.bfloat16)

    @pl.when(s == _E)
    def _():
        out_ref[0] = acc_ref[...]


def kernel(x, w1, w2, gate_w):
    x = x.astype(jnp.float32)
    out = pl.pallas_call(
        _moe_kernel,
        grid=(_E + 1,),
        in_specs=[
            pl.BlockSpec((1, _T, _D), lambda s: (0, 0, 0)),
            pl.BlockSpec((_D, _E), lambda s: (0, 0)),
            pl.BlockSpec((1, _D, _F), lambda s: (jnp.maximum(s - 1, 0), 0, 0)),
            pl.BlockSpec((1, _F, _D), lambda s: (jnp.maximum(s - 1, 0), 0, 0)),
        ],
        out_specs=pl.BlockSpec((1, _T, _D), lambda s: (0, 0, 0)),
        out_shape=jax.ShapeDtypeStruct((1, _T, _D), jnp.bfloat16),
        scratch_shapes=[
            pltpu.VMEM((2 * _E, _T), jnp.float32),
            pltpu.VMEM((2, _K, _D), jnp.bfloat16),
            pltpu.VMEM((2, _K, _T), jnp.bfloat16),
            pltpu.VMEM((_T, _D), jnp.bfloat16),
        ],
        compiler_params=pltpu.CompilerParams(
            dimension_semantics=("arbitrary",),
            vmem_limit_bytes=64 * 1024 * 1024,
        ),
    )(x, gate_w, w1, w2)
    return out
